# baseline (speedup 1.0000x reference)
import functools

import jax
import jax.numpy as jnp
from jax import lax
from jax.experimental import pallas as pl
from jax.experimental.pallas import tpu as pltpu
from jax.experimental.pallas import tpu_sc as plsc

N = 10000
E = 320000
NC, NS, L = 2, 16, 16
NW = NC * NS
CW = 128
NCH = (E // NW + CW - 1) // CW
EWP = NCH * CW
EP = NW * EWP
NPAD = 10240
PER = NPAD // NS
BLK = 1000
GRID = N // BLK


def _make_deg_kernel():
    mesh = plsc.VectorSubcoreMesh(core_axis_name="c", subcore_axis_name="s")

    @functools.partial(
        pl.kernel, mesh=mesh,
        out_type=jax.ShapeDtypeStruct((NC, NPAD), jnp.float32),
        scratch_types=[
            pltpu.VMEM_SHARED((NPAD,), jnp.float32),
            pltpu.VMEM((NCH, CW), jnp.int32),
            pltpu.VMEM((NCH, CW), jnp.float32),
            pltpu.VMEM((PER,), jnp.float32),
        ],
    )
    def deg_kernel(col_hbm, w_hbm, out_hbm, acc, col_v, w_v, init_v):
        c = lax.axis_index("c")
        s = lax.axis_index("s")
        wid = c * NS + s
        val = jnp.where(c == 0, jnp.float32(1.0), jnp.float32(0.0))
        vec = jnp.full((L,), val, jnp.float32)
        for k in range(PER // L):
            init_v[pl.ds(k * L, L)] = vec
        pltpu.sync_copy(init_v, acc.at[pl.ds(s * PER, PER)])
        plsc.subcore_barrier()
        pltpu.sync_copy(col_hbm.at[wid], col_v)
        pltpu.sync_copy(w_hbm.at[wid], w_v)

        def chunk(j, carry):
            pltpu.sync_copy(w_v.at[j], acc.at[col_v.at[j]], add=True)
            return carry

        lax.fori_loop(0, NCH, chunk, 0)
        plsc.subcore_barrier()
        pltpu.sync_copy(acc.at[pl.ds(s * PER, PER)],
                        out_hbm.at[c, pl.ds(s * PER, PER)])

    return deg_kernel


def _make_msg_kernel(F):
    mesh = plsc.VectorSubcoreMesh(core_axis_name="c", subcore_axis_name="s")
    KV = F // L
    SW = 32 if F == 128 else 128
    NBUF = 6 if F == 128 else 4
    PF = 2
    SCH = EWP // SW
    MAIN = (SCH // NBUF) * NBUF
    QV = SW // L
    RPC = CW // SW
    tiled = F == 128

    @functools.partial(
        pl.kernel, mesh=mesh,
        out_type=jax.ShapeDtypeStruct((NC, NPAD, F), jnp.float32),
        scratch_types=[
            pltpu.VMEM_SHARED((NPAD, F), jnp.float32),
            pltpu.VMEM((NCH, CW), jnp.int32),
            pltpu.VMEM((NCH, CW), jnp.float32),
            pltpu.VMEM((NBUF, SW), jnp.int32),
            pltpu.VMEM((NBUF, SW), jnp.int32),
            pltpu.VMEM((NBUF, SW, F), jnp.float32),
            [pltpu.SemaphoreType.DMA] * NBUF,
            [pltpu.SemaphoreType.DMA] * NBUF,
        ],
        compiler_params=pltpu.CompilerParams(use_tc_tiling_on_sc=tiled),
    )
    def msg_kernel(g_hbm, rc_hbm, w_hbm, out_hbm,
                   acc, rc_v, w_v, ridx, cidx, rows_v, gsems, ssems):
        c = lax.axis_index("c")
        s = lax.axis_index("s")
        wid = c * NS + s
        zv = jnp.zeros((L,), jnp.float32)
        mask = jnp.full((L,), 0xFFFF, jnp.int32)

        def rcpos(j, q):
            if RPC == 1:
                return j, q * L
            return j >> 2, (j & 3) * SW + q * L

        def zrow(e, carry):
            for k in range(KV):
                rows_v[0, e, pl.ds(k * L, L)] = zv
            return carry

        lax.fori_loop(0, SW, zrow, 0)
        for t in range(PER // SW):
            pltpu.sync_copy(rows_v.at[0], acc.at[pl.ds(s * PER + t * SW, SW)])
        plsc.subcore_barrier()

        pltpu.sync_copy(rc_hbm.at[wid], rc_v)
        pltpu.sync_copy(w_hbm.at[wid], w_v)

        def unpack_rows(j, b):
            for q in range(QV):
                r, o = rcpos(j, q)
                v = rc_v[r, pl.ds(o, L)]
                ridx[b, pl.ds(q * L, L)] = v & mask

        def unpack_cols(j, b):
            for q in range(QV):
                r, o = rcpos(j, q)
                v = rc_v[r, pl.ds(o, L)]
                cidx[b, pl.ds(q * L, L)] = lax.shift_right_logical(v, 16)

        def gather(j, b):
            unpack_rows(j, b)
            pltpu.async_copy(g_hbm.at[ridx.at[b]], rows_v.at[b], gsems[b])

        def wait_gather(b):
            pltpu.make_async_copy(
                g_hbm.at[ridx.at[b]], rows_v.at[b], gsems[b]).wait()

        def scatter(j, b):
            unpack_cols(j, b)
            pltpu.async_copy(
                rows_v.at[b], acc.at[cidx.at[b]], ssems[b], add=True)

        def wait_scatter(b):
            pltpu.make_async_copy(
                rows_v.at[b], acc.at[cidx.at[b]], ssems[b]).wait()

        def scale(j, b):
            def group(gq, carry):
                r, o = rcpos(j, gq)
                wch = w_v[r, pl.ds(o, L)]
                for i in range(L):
                    wv = wch[i]
                    e = gq * L + i
                    for k in range(KV):
                        rows_v[b, e, pl.ds(k * L, L)] = (
                            rows_v[b, e, pl.ds(k * L, L)] * wv)
                return carry

            if QV > 2:
                lax.fori_loop(0, QV, group, 0)
            else:
                for gq in range(QV):
                    group(gq, 0)

        def step(j, b):
            wait_gather(b)
            scale(j, b)
            scatter(j, b)
            bn = (b + PF) % NBUF
            if isinstance(j, int):
                if j >= NBUF - PF:
                    wait_scatter(bn)
                if j + PF < SCH:
                    gather(j + PF, bn)
            else:
                @pl.when(j >= NBUF - PF)
                def _():
                    wait_scatter(bn)

                @pl.when(j + PF < SCH)
                def _():
                    gather(j + PF, bn)

        for b in range(PF):
            gather(b, b)

        def body(t, carry):
            j0 = t * NBUF
            for b in range(NBUF):
                step(j0 + b, b)
            return carry

        lax.fori_loop(0, MAIN // NBUF, body, 0)
        for j in range(MAIN, SCH):
            step(j, j % NBUF)
        for j in range(SCH - (NBUF - PF), SCH):
            wait_scatter(j % NBUF)
        plsc.subcore_barrier()
        for t in range(PER // CW):
            pltpu.sync_copy(acc.at[pl.ds(s * PER + t * CW, CW)],
                            out_hbm.at[c, pl.ds(s * PER + t * CW, CW)])

    return msg_kernel


def _dinv_block(deg_ref):
    deg = deg_ref[:, 0:1] + deg_ref[:, 1:2]
    return jnp.where(deg > 0, lax.rsqrt(deg), 0.0)


def _g1_body(x_ref, w_ref, deg_ref, out_ref):
    dinv = _dinv_block(deg_ref)
    out_ref[...] = jnp.dot(x_ref[...], w_ref[...],
                           preferred_element_type=jnp.float32) * dinv


def _g1_call(x, W1, deg2T):
    return pl.pallas_call(
        _g1_body,
        grid=(GRID,),
        in_specs=[
            pl.BlockSpec((BLK, 128), lambda i: (i, 0)),
            pl.BlockSpec((128, 128), lambda i: (0, 0)),
            pl.BlockSpec((BLK, NC), lambda i: (i, 0)),
        ],
        out_specs=pl.BlockSpec((BLK, 128), lambda i: (i, 0)),
        out_shape=jax.ShapeDtypeStruct((N, 128), jnp.float32),
    )(x, W1, deg2T)


def _leaky(h):
    return jnp.where(h > 0, h, 0.01 * h)


def _mid_body(s_ref, g_ref, deg_ref, b_ref, w2_ref, out_ref):
    dinv = _dinv_block(deg_ref)
    a1 = _leaky(dinv * (s_ref[0] + s_ref[1] + g_ref[...]) + b_ref[...])
    out_ref[...] = jnp.dot(a1, w2_ref[...],
                           preferred_element_type=jnp.float32) * dinv


def _mid_call(S1, g1, deg2T, b1r, W2):
    return pl.pallas_call(
        _mid_body,
        grid=(GRID,),
        in_specs=[
            pl.BlockSpec((NC, BLK, 128), lambda i: (0, i, 0)),
            pl.BlockSpec((BLK, 128), lambda i: (i, 0)),
            pl.BlockSpec((BLK, NC), lambda i: (i, 0)),
            pl.BlockSpec((1, 128), lambda i: (0, 0)),
            pl.BlockSpec((128, 32), lambda i: (0, 0)),
        ],
        out_specs=pl.BlockSpec((BLK, 32), lambda i: (i, 0)),
        out_shape=jax.ShapeDtypeStruct((N, 32), jnp.float32),
    )(S1, g1, deg2T, b1r, W2)


def _head_body(s_ref, g_ref, deg_ref, b_ref, wc_ref, bc_ref, out_ref):
    dinv = _dinv_block(deg_ref)
    a2 = _leaky(dinv * (s_ref[0] + s_ref[1] + g_ref[...]) + b_ref[...])
    out_ref[...] = jnp.dot(a2, wc_ref[...],
                           preferred_element_type=jnp.float32) + bc_ref[...]


def _head_call(S2, g2, deg2T, b2r, Wcp, bcp):
    return pl.pallas_call(
        _head_body,
        grid=(GRID,),
        in_specs=[
            pl.BlockSpec((NC, BLK, 32), lambda i: (0, i, 0)),
            pl.BlockSpec((BLK, 32), lambda i: (i, 0)),
            pl.BlockSpec((BLK, NC), lambda i: (i, 0)),
            pl.BlockSpec((1, 32), lambda i: (0, 0)),
            pl.BlockSpec((32, 128), lambda i: (0, 0)),
            pl.BlockSpec((1, 128), lambda i: (0, 0)),
        ],
        out_specs=pl.BlockSpec((BLK, 128), lambda i: (i, 0)),
        out_shape=jax.ShapeDtypeStruct((N, 128), jnp.float32),
    )(S2, g2, deg2T, b2r, Wcp, bcp)


def kernel(x, edge_index, edge_attr, W1, b1, W2, b2, Wc, bc):
    row = edge_index[0]
    col = edge_index[1]
    pade = EP - E
    pad_idx = (jnp.arange(pade, dtype=jnp.int32) * 97) % N
    row_p = jnp.concatenate([row, pad_idx])
    col_p = jnp.concatenate([col, pad_idx])
    w_p = jnp.concatenate([edge_attr, jnp.zeros((pade,), jnp.float32)])
    rc_p = (col_p << 16) | row_p
    rc_m = rc_p.reshape(NW, NCH, CW)
    col_d = col_p.reshape(NW, NCH, CW)
    w_d = w_p.reshape(NW, NCH, CW)

    deg_kernel = _make_deg_kernel()
    msg128 = _make_msg_kernel(128)
    msg32 = _make_msg_kernel(32)

    deg2 = deg_kernel(col_d, w_d)
    deg2T = deg2.T

    g1 = _g1_call(x, W1, deg2T)
    S1 = msg128(g1, rc_m, w_d)
    g2 = _mid_call(S1, g1, deg2T, b1.reshape(1, 128), W2)
    S2 = msg32(g2, rc_m, w_d)
    Wcp = jnp.pad(Wc, ((0, 0), (0, 125)))
    bcp = jnp.pad(bc, (0, 125)).reshape(1, 128)
    outp = _head_call(S2, g2, deg2T, b2.reshape(1, 32), Wcp, bcp)
    return outp[:, :3]


# speedup vs baseline: 1.1308x; 1.1308x over previous
import functools

import jax
import jax.numpy as jnp
from jax import lax
from jax.experimental import pallas as pl
from jax.experimental.pallas import tpu as pltpu
from jax.experimental.pallas import tpu_sc as plsc

N = 10000
E = 320000
NC, NS, L = 2, 16, 16
NW = NC * NS
CW = 128
NCH = (E // NW + CW - 1) // CW
EWP = NCH * CW
EP = NW * EWP
NPAD = 10240
PER = NPAD // NS
BLK = 1000
GRID = N // BLK


def _make_deg_kernel():
    mesh = plsc.VectorSubcoreMesh(core_axis_name="c", subcore_axis_name="s")

    @functools.partial(
        pl.kernel, mesh=mesh,
        out_type=jax.ShapeDtypeStruct((NC, NPAD), jnp.float32),
        scratch_types=[
            pltpu.VMEM_SHARED((NPAD,), jnp.float32),
            pltpu.VMEM((NCH, CW), jnp.int32),
            pltpu.VMEM((NCH, CW), jnp.float32),
            pltpu.VMEM((PER,), jnp.float32),
        ],
    )
    def deg_kernel(col_hbm, w_hbm, out_hbm, acc, col_v, w_v, init_v):
        c = lax.axis_index("c")
        s = lax.axis_index("s")
        wid = c * NS + s
        val = jnp.where(c == 0, jnp.float32(1.0), jnp.float32(0.0))
        vec = jnp.full((L,), val, jnp.float32)
        for k in range(PER // L):
            init_v[pl.ds(k * L, L)] = vec
        pltpu.sync_copy(init_v, acc.at[pl.ds(s * PER, PER)])
        plsc.subcore_barrier()
        pltpu.sync_copy(col_hbm.at[wid], col_v)
        pltpu.sync_copy(w_hbm.at[wid], w_v)

        def chunk(j, carry):
            pltpu.sync_copy(w_v.at[j], acc.at[col_v.at[j]], add=True)
            return carry

        lax.fori_loop(0, NCH, chunk, 0)
        plsc.subcore_barrier()
        pltpu.sync_copy(acc.at[pl.ds(s * PER, PER)],
                        out_hbm.at[c, pl.ds(s * PER, PER)])

    return deg_kernel


def _make_msg_kernel(F):
    mesh = plsc.VectorSubcoreMesh(core_axis_name="c", subcore_axis_name="s")
    KV = F // L
    SW = 32 if F == 128 else 128
    NBUF = 6 if F == 128 else 4
    PF = 3 if F == 128 else 2
    SCH = EWP // SW
    MAIN = (SCH // NBUF) * NBUF
    QV = SW // L
    RPC = CW // SW
    tiled = F == 128

    @functools.partial(
        pl.kernel, mesh=mesh,
        out_type=jax.ShapeDtypeStruct((NC, NPAD, F), jnp.float32),
        scratch_types=[
            pltpu.VMEM_SHARED((NPAD, F), jnp.float32),
            pltpu.VMEM((NCH, CW), jnp.int32),
            pltpu.VMEM((NCH, CW), jnp.float32),
            pltpu.VMEM((NBUF, SW), jnp.int32),
            pltpu.VMEM((NBUF, SW), jnp.int32),
            pltpu.VMEM((NBUF, SW, F), jnp.float32),
            [pltpu.SemaphoreType.DMA] * NBUF,
            [pltpu.SemaphoreType.DMA] * NBUF,
        ],
        compiler_params=pltpu.CompilerParams(use_tc_tiling_on_sc=tiled),
    )
    def msg_kernel(g_hbm, rc_hbm, w_hbm, out_hbm,
                   acc, rc_v, w_v, ridx, cidx, rows_v, gsems, ssems):
        c = lax.axis_index("c")
        s = lax.axis_index("s")
        wid = c * NS + s
        zv = jnp.zeros((L,), jnp.float32)
        mask = jnp.full((L,), 0xFFFF, jnp.int32)

        def rcpos(j, q):
            if RPC == 1:
                return j, q * L
            return j >> 2, (j & 3) * SW + q * L

        def zrow(e, carry):
            for k in range(KV):
                rows_v[0, e, pl.ds(k * L, L)] = zv
            return carry

        lax.fori_loop(0, SW, zrow, 0)
        for t in range(PER // SW):
            pltpu.sync_copy(rows_v.at[0], acc.at[pl.ds(s * PER + t * SW, SW)])
        plsc.subcore_barrier()

        pltpu.sync_copy(rc_hbm.at[wid], rc_v)
        pltpu.sync_copy(w_hbm.at[wid], w_v)

        def unpack_rows(j, b):
            for q in range(QV):
                r, o = rcpos(j, q)
                v = rc_v[r, pl.ds(o, L)]
                ridx[b, pl.ds(q * L, L)] = v & mask

        def unpack_cols(j, b):
            for q in range(QV):
                r, o = rcpos(j, q)
                v = rc_v[r, pl.ds(o, L)]
                cidx[b, pl.ds(q * L, L)] = lax.shift_right_logical(v, 16)

        def gather(j, b):
            unpack_rows(j, b)
            pltpu.async_copy(g_hbm.at[ridx.at[b]], rows_v.at[b], gsems[b])

        def wait_gather(b):
            pltpu.make_async_copy(
                g_hbm.at[ridx.at[b]], rows_v.at[b], gsems[b]).wait()

        def scatter(j, b):
            unpack_cols(j, b)
            pltpu.async_copy(
                rows_v.at[b], acc.at[cidx.at[b]], ssems[b], add=True)

        def wait_scatter(b):
            pltpu.make_async_copy(
                rows_v.at[b], acc.at[cidx.at[b]], ssems[b]).wait()

        def scale(j, b):
            def group(gq, carry):
                r, o = rcpos(j, gq)
                wch = w_v[r, pl.ds(o, L)]
                for i in range(L):
                    wv = wch[i]
                    e = gq * L + i
                    for k in range(KV):
                        rows_v[b, e, pl.ds(k * L, L)] = (
                            rows_v[b, e, pl.ds(k * L, L)] * wv)
                return carry

            if QV > 2:
                lax.fori_loop(0, QV, group, 0)
            else:
                for gq in range(QV):
                    group(gq, 0)

        def step(j, b):
            wait_gather(b)
            scale(j, b)
            scatter(j, b)
            bn = (b + PF) % NBUF
            if isinstance(j, int):
                if j >= NBUF - PF:
                    wait_scatter(bn)
                if j + PF < SCH:
                    gather(j + PF, bn)
            else:
                @pl.when(j >= NBUF - PF)
                def _():
                    wait_scatter(bn)

                @pl.when(j + PF < SCH)
                def _():
                    gather(j + PF, bn)

        for b in range(PF):
            gather(b, b)

        def body(t, carry):
            j0 = t * NBUF
            for b in range(NBUF):
                step(j0 + b, b)
            return carry

        lax.fori_loop(0, MAIN // NBUF, body, 0)
        for j in range(MAIN, SCH):
            step(j, j % NBUF)
        for j in range(SCH - (NBUF - PF), SCH):
            wait_scatter(j % NBUF)
        plsc.subcore_barrier()
        for t in range(PER // CW):
            pltpu.sync_copy(acc.at[pl.ds(s * PER + t * CW, CW)],
                            out_hbm.at[c, pl.ds(s * PER + t * CW, CW)])

    return msg_kernel


def _dinv_block(deg_ref):
    deg = deg_ref[:, 0:1] + deg_ref[:, 1:2]
    return jnp.where(deg > 0, lax.rsqrt(deg), 0.0)


def _g1_body(x_ref, w_ref, deg_ref, out_ref):
    dinv = _dinv_block(deg_ref)
    out_ref[...] = jnp.dot(x_ref[...], w_ref[...],
                           preferred_element_type=jnp.float32) * dinv


def _g1_call(x, W1, deg2T):
    return pl.pallas_call(
        _g1_body,
        grid=(GRID,),
        in_specs=[
            pl.BlockSpec((BLK, 128), lambda i: (i, 0)),
            pl.BlockSpec((128, 128), lambda i: (0, 0)),
            pl.BlockSpec((BLK, NC), lambda i: (i, 0)),
        ],
        out_specs=pl.BlockSpec((BLK, 128), lambda i: (i, 0)),
        out_shape=jax.ShapeDtypeStruct((N, 128), jnp.float32),
    )(x, W1, deg2T)


def _leaky(h):
    return jnp.where(h > 0, h, 0.01 * h)


def _mid_body(s_ref, g_ref, deg_ref, b_ref, w2_ref, out_ref):
    dinv = _dinv_block(deg_ref)
    a1 = _leaky(dinv * (s_ref[0] + s_ref[1] + g_ref[...]) + b_ref[...])
    out_ref[...] = jnp.dot(a1, w2_ref[...],
                           preferred_element_type=jnp.float32) * dinv


def _mid_call(S1, g1, deg2T, b1r, W2):
    return pl.pallas_call(
        _mid_body,
        grid=(GRID,),
        in_specs=[
            pl.BlockSpec((NC, BLK, 128), lambda i: (0, i, 0)),
            pl.BlockSpec((BLK, 128), lambda i: (i, 0)),
            pl.BlockSpec((BLK, NC), lambda i: (i, 0)),
            pl.BlockSpec((1, 128), lambda i: (0, 0)),
            pl.BlockSpec((128, 32), lambda i: (0, 0)),
        ],
        out_specs=pl.BlockSpec((BLK, 32), lambda i: (i, 0)),
        out_shape=jax.ShapeDtypeStruct((N, 32), jnp.float32),
    )(S1, g1, deg2T, b1r, W2)


def _head_body(s_ref, g_ref, deg_ref, b_ref, wc_ref, bc_ref, out_ref):
    dinv = _dinv_block(deg_ref)
    a2 = _leaky(dinv * (s_ref[0] + s_ref[1] + g_ref[...]) + b_ref[...])
    out_ref[...] = jnp.dot(a2, wc_ref[...],
                           preferred_element_type=jnp.float32) + bc_ref[...]


def _head_call(S2, g2, deg2T, b2r, Wcp, bcp):
    return pl.pallas_call(
        _head_body,
        grid=(GRID,),
        in_specs=[
            pl.BlockSpec((NC, BLK, 32), lambda i: (0, i, 0)),
            pl.BlockSpec((BLK, 32), lambda i: (i, 0)),
            pl.BlockSpec((BLK, NC), lambda i: (i, 0)),
            pl.BlockSpec((1, 32), lambda i: (0, 0)),
            pl.BlockSpec((32, 8), lambda i: (0, 0)),
            pl.BlockSpec((1, 8), lambda i: (0, 0)),
        ],
        out_specs=pl.BlockSpec((BLK, 8), lambda i: (i, 0)),
        out_shape=jax.ShapeDtypeStruct((N, 8), jnp.float32),
    )(S2, g2, deg2T, b2r, Wcp, bcp)


def kernel(x, edge_index, edge_attr, W1, b1, W2, b2, Wc, bc):
    row = edge_index[0]
    col = edge_index[1]
    pade = EP - E
    pad_idx = (jnp.arange(pade, dtype=jnp.int32) * 97) % N
    row_p = jnp.concatenate([row, pad_idx])
    col_p = jnp.concatenate([col, pad_idx])
    w_p = jnp.concatenate([edge_attr, jnp.zeros((pade,), jnp.float32)])
    rc_p = (col_p << 16) | row_p
    rc_m = rc_p.reshape(NW, NCH, CW)
    col_d = col_p.reshape(NW, NCH, CW)
    w_d = w_p.reshape(NW, NCH, CW)

    deg_kernel = _make_deg_kernel()
    msg128 = _make_msg_kernel(128)
    msg32 = _make_msg_kernel(32)

    deg2 = deg_kernel(col_d, w_d)
    deg2T = deg2.T

    g1 = _g1_call(x, W1, deg2T)
    S1 = msg128(g1, rc_m, w_d)
    g2 = _mid_call(S1, g1, deg2T, b1.reshape(1, 128), W2)
    S2 = msg32(g2, rc_m, w_d)
    Wcp = jnp.pad(Wc, ((0, 0), (0, 5)))
    bcp = jnp.pad(bc, (0, 5)).reshape(1, 8)
    outp = _head_call(S2, g2, deg2T, b2.reshape(1, 32), Wcp, bcp)
    return outp[:, :3]


# speedup vs baseline: 1.1785x; 1.0421x over previous
import functools

import jax
import jax.numpy as jnp
from jax import lax
from jax.experimental import pallas as pl
from jax.experimental.pallas import tpu as pltpu
from jax.experimental.pallas import tpu_sc as plsc

N = 10000
E = 320000
NC, NS, L = 2, 16, 16
NW = NC * NS
CW = 128
NCH = (E // NW + CW - 1) // CW
EWP = NCH * CW
EP = NW * EWP
NPAD = 10240
PER = NPAD // NS
BLK = 1000
GRID = N // BLK


def _make_deg_kernel():
    mesh = plsc.VectorSubcoreMesh(core_axis_name="c", subcore_axis_name="s")

    @functools.partial(
        pl.kernel, mesh=mesh,
        out_type=jax.ShapeDtypeStruct((NC, NPAD), jnp.float32),
        scratch_types=[
            pltpu.VMEM_SHARED((NPAD,), jnp.float32),
            pltpu.VMEM((NCH, CW), jnp.int32),
            pltpu.VMEM((NCH, CW), jnp.float32),
            pltpu.VMEM((PER,), jnp.float32),
        ],
    )
    def deg_kernel(col_hbm, w_hbm, out_hbm, acc, col_v, w_v, init_v):
        c = lax.axis_index("c")
        s = lax.axis_index("s")
        wid = c * NS + s
        val = jnp.where(c == 0, jnp.float32(1.0), jnp.float32(0.0))
        vec = jnp.full((L,), val, jnp.float32)
        for k in range(PER // L):
            init_v[pl.ds(k * L, L)] = vec
        pltpu.sync_copy(init_v, acc.at[pl.ds(s * PER, PER)])
        plsc.subcore_barrier()
        pltpu.sync_copy(col_hbm.at[wid], col_v)
        pltpu.sync_copy(w_hbm.at[wid], w_v)

        def chunk(j, carry):
            pltpu.sync_copy(w_v.at[j], acc.at[col_v.at[j]], add=True)
            return carry

        lax.fori_loop(0, NCH, chunk, 0)
        plsc.subcore_barrier()
        pltpu.sync_copy(acc.at[pl.ds(s * PER, PER)],
                        out_hbm.at[c, pl.ds(s * PER, PER)])

    return deg_kernel


def _make_msg_kernel(F):
    mesh = plsc.VectorSubcoreMesh(core_axis_name="c", subcore_axis_name="s")
    KV = F // L
    SW = 32 if F == 128 else 128
    NBUF = 6 if F == 128 else 4
    PF = 4 if F == 128 else 2
    SCH = EWP // SW
    MAIN = (SCH // NBUF) * NBUF
    QV = SW // L
    RPC = CW // SW
    tiled = F == 128

    @functools.partial(
        pl.kernel, mesh=mesh,
        out_type=jax.ShapeDtypeStruct((NC, NPAD, F), jnp.float32),
        scratch_types=[
            pltpu.VMEM_SHARED((NPAD, F), jnp.float32),
            pltpu.VMEM((NCH, CW), jnp.int32),
            pltpu.VMEM((NCH, CW), jnp.float32),
            pltpu.VMEM((NBUF, SW), jnp.int32),
            pltpu.VMEM((NBUF, SW), jnp.int32),
            pltpu.VMEM((NBUF, SW, F), jnp.float32),
            [pltpu.SemaphoreType.DMA] * NBUF,
            [pltpu.SemaphoreType.DMA] * NBUF,
        ],
        compiler_params=pltpu.CompilerParams(use_tc_tiling_on_sc=tiled),
    )
    def msg_kernel(g_hbm, rc_hbm, w_hbm, out_hbm,
                   acc, rc_v, w_v, ridx, cidx, rows_v, gsems, ssems):
        c = lax.axis_index("c")
        s = lax.axis_index("s")
        wid = c * NS + s
        zv = jnp.zeros((L,), jnp.float32)
        mask = jnp.full((L,), 0xFFFF, jnp.int32)

        def rcpos(j, q):
            if RPC == 1:
                return j, q * L
            return j >> 2, (j & 3) * SW + q * L

        def zrow(e, carry):
            for k in range(KV):
                rows_v[0, e, pl.ds(k * L, L)] = zv
            return carry

        lax.fori_loop(0, SW, zrow, 0)
        for t in range(PER // SW):
            pltpu.sync_copy(rows_v.at[0], acc.at[pl.ds(s * PER + t * SW, SW)])
        plsc.subcore_barrier()

        pltpu.sync_copy(rc_hbm.at[wid], rc_v)
        pltpu.sync_copy(w_hbm.at[wid], w_v)

        def unpack_rows(j, b):
            for q in range(QV):
                r, o = rcpos(j, q)
                v = rc_v[r, pl.ds(o, L)]
                ridx[b, pl.ds(q * L, L)] = v & mask

        def unpack_cols(j, b):
            for q in range(QV):
                r, o = rcpos(j, q)
                v = rc_v[r, pl.ds(o, L)]
                cidx[b, pl.ds(q * L, L)] = lax.shift_right_logical(v, 16)

        def gather(j, b):
            unpack_rows(j, b)
            pltpu.async_copy(g_hbm.at[ridx.at[b]], rows_v.at[b], gsems[b])

        def wait_gather(b):
            pltpu.make_async_copy(
                g_hbm.at[ridx.at[b]], rows_v.at[b], gsems[b]).wait()

        def scatter(j, b):
            unpack_cols(j, b)
            pltpu.async_copy(
                rows_v.at[b], acc.at[cidx.at[b]], ssems[b], add=True)

        def wait_scatter(b):
            pltpu.make_async_copy(
                rows_v.at[b], acc.at[cidx.at[b]], ssems[b]).wait()

        def scale(j, b):
            def group(gq, carry):
                r, o = rcpos(j, gq)
                wch = w_v[r, pl.ds(o, L)]
                for i in range(L):
                    wv = wch[i]
                    e = gq * L + i
                    for k in range(KV):
                        rows_v[b, e, pl.ds(k * L, L)] = (
                            rows_v[b, e, pl.ds(k * L, L)] * wv)
                return carry

            if QV > 2:
                lax.fori_loop(0, QV, group, 0)
            else:
                for gq in range(QV):
                    group(gq, 0)

        def step(j, b):
            wait_gather(b)
            scale(j, b)
            scatter(j, b)
            bn = (b + PF) % NBUF
            if isinstance(j, int):
                if j >= NBUF - PF:
                    wait_scatter(bn)
                if j + PF < SCH:
                    gather(j + PF, bn)
            else:
                @pl.when(j >= NBUF - PF)
                def _():
                    wait_scatter(bn)

                @pl.when(j + PF < SCH)
                def _():
                    gather(j + PF, bn)

        for b in range(PF):
            gather(b, b)

        def body(t, carry):
            j0 = t * NBUF
            for b in range(NBUF):
                step(j0 + b, b)
            return carry

        lax.fori_loop(0, MAIN // NBUF, body, 0)
        for j in range(MAIN, SCH):
            step(j, j % NBUF)
        for j in range(SCH - (NBUF - PF), SCH):
            wait_scatter(j % NBUF)
        plsc.subcore_barrier()
        for t in range(PER // CW):
            pltpu.sync_copy(acc.at[pl.ds(s * PER + t * CW, CW)],
                            out_hbm.at[c, pl.ds(s * PER + t * CW, CW)])

    return msg_kernel


def _dinv_block(deg_ref):
    deg = deg_ref[:, 0:1] + deg_ref[:, 1:2]
    return jnp.where(deg > 0, lax.rsqrt(deg), 0.0)


def _g1_body(x_ref, w_ref, deg_ref, out_ref):
    dinv = _dinv_block(deg_ref)
    out_ref[...] = jnp.dot(x_ref[...], w_ref[...],
                           preferred_element_type=jnp.float32) * dinv


def _g1_call(x, W1, deg2T):
    return pl.pallas_call(
        _g1_body,
        grid=(GRID,),
        in_specs=[
            pl.BlockSpec((BLK, 128), lambda i: (i, 0)),
            pl.BlockSpec((128, 128), lambda i: (0, 0)),
            pl.BlockSpec((BLK, NC), lambda i: (i, 0)),
        ],
        out_specs=pl.BlockSpec((BLK, 128), lambda i: (i, 0)),
        out_shape=jax.ShapeDtypeStruct((N, 128), jnp.float32),
    )(x, W1, deg2T)


def _leaky(h):
    return jnp.where(h > 0, h, 0.01 * h)


def _mid_body(s_ref, g_ref, deg_ref, b_ref, w2_ref, out_ref):
    dinv = _dinv_block(deg_ref)
    a1 = _leaky(dinv * (s_ref[0] + s_ref[1] + g_ref[...]) + b_ref[...])
    out_ref[...] = jnp.dot(a1, w2_ref[...],
                           preferred_element_type=jnp.float32) * dinv


def _mid_call(S1, g1, deg2T, b1r, W2):
    return pl.pallas_call(
        _mid_body,
        grid=(GRID,),
        in_specs=[
            pl.BlockSpec((NC, BLK, 128), lambda i: (0, i, 0)),
            pl.BlockSpec((BLK, 128), lambda i: (i, 0)),
            pl.BlockSpec((BLK, NC), lambda i: (i, 0)),
            pl.BlockSpec((1, 128), lambda i: (0, 0)),
            pl.BlockSpec((128, 32), lambda i: (0, 0)),
        ],
        out_specs=pl.BlockSpec((BLK, 32), lambda i: (i, 0)),
        out_shape=jax.ShapeDtypeStruct((N, 32), jnp.float32),
    )(S1, g1, deg2T, b1r, W2)


def _head_body(s_ref, g_ref, deg_ref, b_ref, wc_ref, bc_ref, out_ref):
    dinv = _dinv_block(deg_ref)
    a2 = _leaky(dinv * (s_ref[0] + s_ref[1] + g_ref[...]) + b_ref[...])
    out_ref[...] = jnp.dot(a2, wc_ref[...],
                           preferred_element_type=jnp.float32) + bc_ref[...]


def _head_call(S2, g2, deg2T, b2r, Wcp, bcp):
    return pl.pallas_call(
        _head_body,
        grid=(GRID,),
        in_specs=[
            pl.BlockSpec((NC, BLK, 32), lambda i: (0, i, 0)),
            pl.BlockSpec((BLK, 32), lambda i: (i, 0)),
            pl.BlockSpec((BLK, NC), lambda i: (i, 0)),
            pl.BlockSpec((1, 32), lambda i: (0, 0)),
            pl.BlockSpec((32, 8), lambda i: (0, 0)),
            pl.BlockSpec((1, 8), lambda i: (0, 0)),
        ],
        out_specs=pl.BlockSpec((BLK, 8), lambda i: (i, 0)),
        out_shape=jax.ShapeDtypeStruct((N, 8), jnp.float32),
    )(S2, g2, deg2T, b2r, Wcp, bcp)


def kernel(x, edge_index, edge_attr, W1, b1, W2, b2, Wc, bc):
    row = edge_index[0]
    col = edge_index[1]
    pade = EP - E
    pad_idx = (jnp.arange(pade, dtype=jnp.int32) * 97) % N
    row_p = jnp.concatenate([row, pad_idx])
    col_p = jnp.concatenate([col, pad_idx])
    w_p = jnp.concatenate([edge_attr, jnp.zeros((pade,), jnp.float32)])
    rc_p = (col_p << 16) | row_p
    rc_m = rc_p.reshape(NW, NCH, CW)
    col_d = col_p.reshape(NW, NCH, CW)
    w_d = w_p.reshape(NW, NCH, CW)

    deg_kernel = _make_deg_kernel()
    msg128 = _make_msg_kernel(128)
    msg32 = _make_msg_kernel(32)

    deg2 = deg_kernel(col_d, w_d)
    deg2T = deg2.T

    g1 = _g1_call(x, W1, deg2T)
    S1 = msg128(g1, rc_m, w_d)
    g2 = _mid_call(S1, g1, deg2T, b1.reshape(1, 128), W2)
    S2 = msg32(g2, rc_m, w_d)
    Wcp = jnp.pad(Wc, ((0, 0), (0, 5)))
    bcp = jnp.pad(bc, (0, 5)).reshape(1, 8)
    outp = _head_call(S2, g2, deg2T, b2.reshape(1, 32), Wcp, bcp)
    return outp[:, :3]


# speedup vs baseline: 1.2038x; 1.0215x over previous
import functools

import jax
import jax.numpy as jnp
from jax import lax
from jax.experimental import pallas as pl
from jax.experimental.pallas import tpu as pltpu
from jax.experimental.pallas import tpu_sc as plsc

N = 10000
E = 320000
NC, NS, L = 2, 16, 16
NW = NC * NS
CW = 128
NCH = (E // NW + CW - 1) // CW
EWP = NCH * CW
EP = NW * EWP
NPAD = 10240
PER = NPAD // NS
BLK = 1000
GRID = N // BLK


def _make_deg_kernel():
    mesh = plsc.VectorSubcoreMesh(core_axis_name="c", subcore_axis_name="s")

    @functools.partial(
        pl.kernel, mesh=mesh,
        out_type=jax.ShapeDtypeStruct((NC, NPAD), jnp.float32),
        scratch_types=[
            pltpu.VMEM_SHARED((NPAD,), jnp.float32),
            pltpu.VMEM((NCH, CW), jnp.int32),
            pltpu.VMEM((NCH, CW), jnp.float32),
            pltpu.VMEM((PER,), jnp.float32),
        ],
    )
    def deg_kernel(col_hbm, w_hbm, out_hbm, acc, col_v, w_v, init_v):
        c = lax.axis_index("c")
        s = lax.axis_index("s")
        wid = c * NS + s
        val = jnp.where(c == 0, jnp.float32(1.0), jnp.float32(0.0))
        vec = jnp.full((L,), val, jnp.float32)
        for k in range(PER // L):
            init_v[pl.ds(k * L, L)] = vec
        pltpu.sync_copy(init_v, acc.at[pl.ds(s * PER, PER)])
        plsc.subcore_barrier()
        pltpu.sync_copy(col_hbm.at[wid], col_v)
        pltpu.sync_copy(w_hbm.at[wid], w_v)

        def chunk(j, carry):
            pltpu.sync_copy(w_v.at[j], acc.at[col_v.at[j]], add=True)
            return carry

        lax.fori_loop(0, NCH, chunk, 0)
        plsc.subcore_barrier()
        pltpu.sync_copy(acc.at[pl.ds(s * PER, PER)],
                        out_hbm.at[c, pl.ds(s * PER, PER)])

    return deg_kernel


def _make_msg_kernel(F):
    mesh = plsc.VectorSubcoreMesh(core_axis_name="c", subcore_axis_name="s")
    KV = F // L
    SW = 32 if F == 128 else 128
    NBUF = 6 if F == 128 else 4
    PF = 5 if F == 128 else 2
    SCH = EWP // SW
    MAIN = (SCH // NBUF) * NBUF
    QV = SW // L
    RPC = CW // SW
    tiled = F == 128

    @functools.partial(
        pl.kernel, mesh=mesh,
        out_type=jax.ShapeDtypeStruct((NC, NPAD, F), jnp.float32),
        scratch_types=[
            pltpu.VMEM_SHARED((NPAD, F), jnp.float32),
            pltpu.VMEM((NCH, CW), jnp.int32),
            pltpu.VMEM((NCH, CW), jnp.float32),
            pltpu.VMEM((NBUF, SW), jnp.int32),
            pltpu.VMEM((NBUF, SW), jnp.int32),
            pltpu.VMEM((NBUF, SW, F), jnp.float32),
            [pltpu.SemaphoreType.DMA] * NBUF,
            [pltpu.SemaphoreType.DMA] * NBUF,
        ],
        compiler_params=pltpu.CompilerParams(use_tc_tiling_on_sc=tiled),
    )
    def msg_kernel(g_hbm, rc_hbm, w_hbm, out_hbm,
                   acc, rc_v, w_v, ridx, cidx, rows_v, gsems, ssems):
        c = lax.axis_index("c")
        s = lax.axis_index("s")
        wid = c * NS + s
        zv = jnp.zeros((L,), jnp.float32)
        mask = jnp.full((L,), 0xFFFF, jnp.int32)

        def rcpos(j, q):
            if RPC == 1:
                return j, q * L
            return j >> 2, (j & 3) * SW + q * L

        def zrow(e, carry):
            for k in range(KV):
                rows_v[0, e, pl.ds(k * L, L)] = zv
            return carry

        lax.fori_loop(0, SW, zrow, 0)
        for t in range(PER // SW):
            pltpu.sync_copy(rows_v.at[0], acc.at[pl.ds(s * PER + t * SW, SW)])
        plsc.subcore_barrier()

        pltpu.sync_copy(rc_hbm.at[wid], rc_v)
        pltpu.sync_copy(w_hbm.at[wid], w_v)

        def unpack_rows(j, b):
            for q in range(QV):
                r, o = rcpos(j, q)
                v = rc_v[r, pl.ds(o, L)]
                ridx[b, pl.ds(q * L, L)] = v & mask

        def unpack_cols(j, b):
            for q in range(QV):
                r, o = rcpos(j, q)
                v = rc_v[r, pl.ds(o, L)]
                cidx[b, pl.ds(q * L, L)] = lax.shift_right_logical(v, 16)

        def gather(j, b):
            unpack_rows(j, b)
            pltpu.async_copy(g_hbm.at[ridx.at[b]], rows_v.at[b], gsems[b])

        def wait_gather(b):
            pltpu.make_async_copy(
                g_hbm.at[ridx.at[b]], rows_v.at[b], gsems[b]).wait()

        def scatter(j, b):
            unpack_cols(j, b)
            pltpu.async_copy(
                rows_v.at[b], acc.at[cidx.at[b]], ssems[b], add=True)

        def wait_scatter(b):
            pltpu.make_async_copy(
                rows_v.at[b], acc.at[cidx.at[b]], ssems[b]).wait()

        def scale(j, b):
            def group(gq, carry):
                r, o = rcpos(j, gq)
                wch = w_v[r, pl.ds(o, L)]
                for i in range(L):
                    wv = wch[i]
                    e = gq * L + i
                    for k in range(KV):
                        rows_v[b, e, pl.ds(k * L, L)] = (
                            rows_v[b, e, pl.ds(k * L, L)] * wv)
                return carry

            if QV > 2:
                lax.fori_loop(0, QV, group, 0)
            else:
                for gq in range(QV):
                    group(gq, 0)

        def step(j, b):
            wait_gather(b)
            scale(j, b)
            scatter(j, b)
            bn = (b + PF) % NBUF
            if isinstance(j, int):
                if j >= NBUF - PF:
                    wait_scatter(bn)
                if j + PF < SCH:
                    gather(j + PF, bn)
            else:
                @pl.when(j >= NBUF - PF)
                def _():
                    wait_scatter(bn)

                @pl.when(j + PF < SCH)
                def _():
                    gather(j + PF, bn)

        for b in range(PF):
            gather(b, b)

        def body(t, carry):
            j0 = t * NBUF
            for b in range(NBUF):
                step(j0 + b, b)
            return carry

        lax.fori_loop(0, MAIN // NBUF, body, 0)
        for j in range(MAIN, SCH):
            step(j, j % NBUF)
        for j in range(SCH - (NBUF - PF), SCH):
            wait_scatter(j % NBUF)
        plsc.subcore_barrier()
        for t in range(PER // CW):
            pltpu.sync_copy(acc.at[pl.ds(s * PER + t * CW, CW)],
                            out_hbm.at[c, pl.ds(s * PER + t * CW, CW)])

    return msg_kernel


def _dinv_block(deg_ref):
    deg = deg_ref[:, 0:1] + deg_ref[:, 1:2]
    return jnp.where(deg > 0, lax.rsqrt(deg), 0.0)


def _g1_body(x_ref, w_ref, deg_ref, out_ref):
    dinv = _dinv_block(deg_ref)
    out_ref[...] = jnp.dot(x_ref[...], w_ref[...],
                           preferred_element_type=jnp.float32) * dinv


def _g1_call(x, W1, deg2T):
    return pl.pallas_call(
        _g1_body,
        grid=(GRID,),
        in_specs=[
            pl.BlockSpec((BLK, 128), lambda i: (i, 0)),
            pl.BlockSpec((128, 128), lambda i: (0, 0)),
            pl.BlockSpec((BLK, NC), lambda i: (i, 0)),
        ],
        out_specs=pl.BlockSpec((BLK, 128), lambda i: (i, 0)),
        out_shape=jax.ShapeDtypeStruct((N, 128), jnp.float32),
    )(x, W1, deg2T)


def _leaky(h):
    return jnp.where(h > 0, h, 0.01 * h)


def _mid_body(s_ref, g_ref, deg_ref, b_ref, w2_ref, out_ref):
    dinv = _dinv_block(deg_ref)
    a1 = _leaky(dinv * (s_ref[0] + s_ref[1] + g_ref[...]) + b_ref[...])
    out_ref[...] = jnp.dot(a1, w2_ref[...],
                           preferred_element_type=jnp.float32) * dinv


def _mid_call(S1, g1, deg2T, b1r, W2):
    return pl.pallas_call(
        _mid_body,
        grid=(GRID,),
        in_specs=[
            pl.BlockSpec((NC, BLK, 128), lambda i: (0, i, 0)),
            pl.BlockSpec((BLK, 128), lambda i: (i, 0)),
            pl.BlockSpec((BLK, NC), lambda i: (i, 0)),
            pl.BlockSpec((1, 128), lambda i: (0, 0)),
            pl.BlockSpec((128, 32), lambda i: (0, 0)),
        ],
        out_specs=pl.BlockSpec((BLK, 32), lambda i: (i, 0)),
        out_shape=jax.ShapeDtypeStruct((N, 32), jnp.float32),
    )(S1, g1, deg2T, b1r, W2)


def _head_body(s_ref, g_ref, deg_ref, b_ref, wc_ref, bc_ref, out_ref):
    dinv = _dinv_block(deg_ref)
    a2 = _leaky(dinv * (s_ref[0] + s_ref[1] + g_ref[...]) + b_ref[...])
    out_ref[...] = jnp.dot(a2, wc_ref[...],
                           preferred_element_type=jnp.float32) + bc_ref[...]


def _head_call(S2, g2, deg2T, b2r, Wcp, bcp):
    return pl.pallas_call(
        _head_body,
        grid=(GRID,),
        in_specs=[
            pl.BlockSpec((NC, BLK, 32), lambda i: (0, i, 0)),
            pl.BlockSpec((BLK, 32), lambda i: (i, 0)),
            pl.BlockSpec((BLK, NC), lambda i: (i, 0)),
            pl.BlockSpec((1, 32), lambda i: (0, 0)),
            pl.BlockSpec((32, 8), lambda i: (0, 0)),
            pl.BlockSpec((1, 8), lambda i: (0, 0)),
        ],
        out_specs=pl.BlockSpec((BLK, 8), lambda i: (i, 0)),
        out_shape=jax.ShapeDtypeStruct((N, 8), jnp.float32),
    )(S2, g2, deg2T, b2r, Wcp, bcp)


def kernel(x, edge_index, edge_attr, W1, b1, W2, b2, Wc, bc):
    row = edge_index[0]
    col = edge_index[1]
    pade = EP - E
    pad_idx = (jnp.arange(pade, dtype=jnp.int32) * 97) % N
    row_p = jnp.concatenate([row, pad_idx])
    col_p = jnp.concatenate([col, pad_idx])
    w_p = jnp.concatenate([edge_attr, jnp.zeros((pade,), jnp.float32)])
    rc_p = (col_p << 16) | row_p
    rc_m = rc_p.reshape(NW, NCH, CW)
    col_d = col_p.reshape(NW, NCH, CW)
    w_d = w_p.reshape(NW, NCH, CW)

    deg_kernel = _make_deg_kernel()
    msg128 = _make_msg_kernel(128)
    msg32 = _make_msg_kernel(32)

    deg2 = deg_kernel(col_d, w_d)
    deg2T = deg2.T

    g1 = _g1_call(x, W1, deg2T)
    S1 = msg128(g1, rc_m, w_d)
    g2 = _mid_call(S1, g1, deg2T, b1.reshape(1, 128), W2)
    S2 = msg32(g2, rc_m, w_d)
    Wcp = jnp.pad(Wc, ((0, 0), (0, 5)))
    bcp = jnp.pad(bc, (0, 5)).reshape(1, 8)
    outp = _head_call(S2, g2, deg2T, b2.reshape(1, 32), Wcp, bcp)
    return outp[:, :3]


# speedup vs baseline: 1.2726x; 1.0572x over previous
import functools

import jax
import jax.numpy as jnp
from jax import lax
from jax.experimental import pallas as pl
from jax.experimental.pallas import tpu as pltpu
from jax.experimental.pallas import tpu_sc as plsc

N = 10000
E = 320000
NC, NS, L = 2, 16, 16
NW = NC * NS
CW = 128
NCH = (E // NW + CW - 1) // CW
EWP = NCH * CW
EP = NW * EWP
NPAD = 10240
PER = NPAD // NS
BLK = 1000
GRID = N // BLK


def _make_deg_kernel():
    mesh = plsc.VectorSubcoreMesh(core_axis_name="c", subcore_axis_name="s")

    @functools.partial(
        pl.kernel, mesh=mesh,
        out_type=jax.ShapeDtypeStruct((NC, NPAD), jnp.float32),
        scratch_types=[
            pltpu.VMEM_SHARED((NPAD,), jnp.float32),
            pltpu.VMEM((NCH, CW), jnp.int32),
            pltpu.VMEM((NCH, CW), jnp.float32),
            pltpu.VMEM((PER,), jnp.float32),
        ],
    )
    def deg_kernel(col_hbm, w_hbm, out_hbm, acc, col_v, w_v, init_v):
        c = lax.axis_index("c")
        s = lax.axis_index("s")
        wid = c * NS + s
        val = jnp.where(c == 0, jnp.float32(1.0), jnp.float32(0.0))
        vec = jnp.full((L,), val, jnp.float32)
        for k in range(PER // L):
            init_v[pl.ds(k * L, L)] = vec
        pltpu.sync_copy(init_v, acc.at[pl.ds(s * PER, PER)])
        plsc.subcore_barrier()
        pltpu.sync_copy(col_hbm.at[wid], col_v)
        pltpu.sync_copy(w_hbm.at[wid], w_v)

        def chunk(j, carry):
            pltpu.sync_copy(w_v.at[j], acc.at[col_v.at[j]], add=True)
            return carry

        lax.fori_loop(0, NCH, chunk, 0)
        plsc.subcore_barrier()
        pltpu.sync_copy(acc.at[pl.ds(s * PER, PER)],
                        out_hbm.at[c, pl.ds(s * PER, PER)])

    return deg_kernel


def _make_msg_kernel(F):
    mesh = plsc.VectorSubcoreMesh(core_axis_name="c", subcore_axis_name="s")
    KV = F // L
    SW = 32 if F == 128 else 128
    NBUF = 6
    PF = 5 if F == 128 else 4
    SCH = EWP // SW
    MAIN = (SCH // NBUF) * NBUF
    QV = SW // L
    RPC = CW // SW
    tiled = F == 128

    @functools.partial(
        pl.kernel, mesh=mesh,
        out_type=jax.ShapeDtypeStruct((NC, NPAD, F), jnp.float32),
        scratch_types=[
            pltpu.VMEM_SHARED((NPAD, F), jnp.float32),
            pltpu.VMEM((NCH, CW), jnp.int32),
            pltpu.VMEM((NCH, CW), jnp.float32),
            pltpu.VMEM((NBUF, SW), jnp.int32),
            pltpu.VMEM((NBUF, SW), jnp.int32),
            pltpu.VMEM((NBUF, SW, F), jnp.float32),
            [pltpu.SemaphoreType.DMA] * NBUF,
            [pltpu.SemaphoreType.DMA] * NBUF,
        ],
        compiler_params=pltpu.CompilerParams(use_tc_tiling_on_sc=tiled),
    )
    def msg_kernel(g_hbm, rc_hbm, w_hbm, out_hbm,
                   acc, rc_v, w_v, ridx, cidx, rows_v, gsems, ssems):
        c = lax.axis_index("c")
        s = lax.axis_index("s")
        wid = c * NS + s
        zv = jnp.zeros((L,), jnp.float32)
        mask = jnp.full((L,), 0xFFFF, jnp.int32)

        def rcpos(j, q):
            if RPC == 1:
                return j, q * L
            return j >> 2, (j & 3) * SW + q * L

        def zrow(e, carry):
            for k in range(KV):
                rows_v[0, e, pl.ds(k * L, L)] = zv
            return carry

        lax.fori_loop(0, SW, zrow, 0)
        for t in range(PER // SW):
            pltpu.sync_copy(rows_v.at[0], acc.at[pl.ds(s * PER + t * SW, SW)])
        plsc.subcore_barrier()

        pltpu.sync_copy(rc_hbm.at[wid], rc_v)
        pltpu.sync_copy(w_hbm.at[wid], w_v)

        def unpack_rows(j, b):
            for q in range(QV):
                r, o = rcpos(j, q)
                v = rc_v[r, pl.ds(o, L)]
                ridx[b, pl.ds(q * L, L)] = v & mask

        def unpack_cols(j, b):
            for q in range(QV):
                r, o = rcpos(j, q)
                v = rc_v[r, pl.ds(o, L)]
                cidx[b, pl.ds(q * L, L)] = lax.shift_right_logical(v, 16)

        def gather(j, b):
            unpack_rows(j, b)
            pltpu.async_copy(g_hbm.at[ridx.at[b]], rows_v.at[b], gsems[b])

        def wait_gather(b):
            pltpu.make_async_copy(
                g_hbm.at[ridx.at[b]], rows_v.at[b], gsems[b]).wait()

        def scatter(j, b):
            unpack_cols(j, b)
            pltpu.async_copy(
                rows_v.at[b], acc.at[cidx.at[b]], ssems[b], add=True)

        def wait_scatter(b):
            pltpu.make_async_copy(
                rows_v.at[b], acc.at[cidx.at[b]], ssems[b]).wait()

        def scale(j, b):
            def group(gq, carry):
                r, o = rcpos(j, gq)
                wch = w_v[r, pl.ds(o, L)]
                for i in range(L):
                    wv = wch[i]
                    e = gq * L + i
                    for k in range(KV):
                        rows_v[b, e, pl.ds(k * L, L)] = (
                            rows_v[b, e, pl.ds(k * L, L)] * wv)
                return carry

            if QV > 2:
                lax.fori_loop(0, QV, group, 0)
            else:
                for gq in range(QV):
                    group(gq, 0)

        def step(j, b):
            wait_gather(b)
            scale(j, b)
            scatter(j, b)
            bn = (b + PF) % NBUF
            if isinstance(j, int):
                if j >= NBUF - PF:
                    wait_scatter(bn)
                if j + PF < SCH:
                    gather(j + PF, bn)
            else:
                @pl.when(j >= NBUF - PF)
                def _():
                    wait_scatter(bn)

                @pl.when(j + PF < SCH)
                def _():
                    gather(j + PF, bn)

        for b in range(PF):
            gather(b, b)

        def body(t, carry):
            j0 = t * NBUF
            for b in range(NBUF):
                step(j0 + b, b)
            return carry

        lax.fori_loop(0, MAIN // NBUF, body, 0)
        for j in range(MAIN, SCH):
            step(j, j % NBUF)
        for j in range(SCH - (NBUF - PF), SCH):
            wait_scatter(j % NBUF)
        plsc.subcore_barrier()
        for t in range(PER // CW):
            pltpu.sync_copy(acc.at[pl.ds(s * PER + t * CW, CW)],
                            out_hbm.at[c, pl.ds(s * PER + t * CW, CW)])

    return msg_kernel


def _dinv_block(deg_ref):
    deg = deg_ref[:, 0:1] + deg_ref[:, 1:2]
    return jnp.where(deg > 0, lax.rsqrt(deg), 0.0)


def _g1_body(x_ref, w_ref, deg_ref, out_ref):
    dinv = _dinv_block(deg_ref)
    out_ref[...] = jnp.dot(x_ref[...], w_ref[...],
                           preferred_element_type=jnp.float32) * dinv


def _g1_call(x, W1, deg2T):
    return pl.pallas_call(
        _g1_body,
        grid=(GRID,),
        in_specs=[
            pl.BlockSpec((BLK, 128), lambda i: (i, 0)),
            pl.BlockSpec((128, 128), lambda i: (0, 0)),
            pl.BlockSpec((BLK, NC), lambda i: (i, 0)),
        ],
        out_specs=pl.BlockSpec((BLK, 128), lambda i: (i, 0)),
        out_shape=jax.ShapeDtypeStruct((N, 128), jnp.float32),
    )(x, W1, deg2T)


def _leaky(h):
    return jnp.where(h > 0, h, 0.01 * h)


def _mid_body(s_ref, g_ref, deg_ref, b_ref, w2_ref, out_ref):
    dinv = _dinv_block(deg_ref)
    a1 = _leaky(dinv * (s_ref[0] + s_ref[1] + g_ref[...]) + b_ref[...])
    out_ref[...] = jnp.dot(a1, w2_ref[...],
                           preferred_element_type=jnp.float32) * dinv


def _mid_call(S1, g1, deg2T, b1r, W2):
    return pl.pallas_call(
        _mid_body,
        grid=(GRID,),
        in_specs=[
            pl.BlockSpec((NC, BLK, 128), lambda i: (0, i, 0)),
            pl.BlockSpec((BLK, 128), lambda i: (i, 0)),
            pl.BlockSpec((BLK, NC), lambda i: (i, 0)),
            pl.BlockSpec((1, 128), lambda i: (0, 0)),
            pl.BlockSpec((128, 32), lambda i: (0, 0)),
        ],
        out_specs=pl.BlockSpec((BLK, 32), lambda i: (i, 0)),
        out_shape=jax.ShapeDtypeStruct((N, 32), jnp.float32),
    )(S1, g1, deg2T, b1r, W2)


def _head_body(s_ref, g_ref, deg_ref, b_ref, wc_ref, bc_ref, out_ref):
    dinv = _dinv_block(deg_ref)
    a2 = _leaky(dinv * (s_ref[0] + s_ref[1] + g_ref[...]) + b_ref[...])
    out_ref[...] = jnp.dot(a2, wc_ref[...],
                           preferred_element_type=jnp.float32) + bc_ref[...]


def _head_call(S2, g2, deg2T, b2r, Wcp, bcp):
    return pl.pallas_call(
        _head_body,
        grid=(GRID,),
        in_specs=[
            pl.BlockSpec((NC, BLK, 32), lambda i: (0, i, 0)),
            pl.BlockSpec((BLK, 32), lambda i: (i, 0)),
            pl.BlockSpec((BLK, NC), lambda i: (i, 0)),
            pl.BlockSpec((1, 32), lambda i: (0, 0)),
            pl.BlockSpec((32, 8), lambda i: (0, 0)),
            pl.BlockSpec((1, 8), lambda i: (0, 0)),
        ],
        out_specs=pl.BlockSpec((BLK, 8), lambda i: (i, 0)),
        out_shape=jax.ShapeDtypeStruct((N, 8), jnp.float32),
    )(S2, g2, deg2T, b2r, Wcp, bcp)


def kernel(x, edge_index, edge_attr, W1, b1, W2, b2, Wc, bc):
    row = edge_index[0]
    col = edge_index[1]
    pade = EP - E
    pad_idx = (jnp.arange(pade, dtype=jnp.int32) * 97) % N
    row_p = jnp.concatenate([row, pad_idx])
    col_p = jnp.concatenate([col, pad_idx])
    w_p = jnp.concatenate([edge_attr, jnp.zeros((pade,), jnp.float32)])
    rc_p = (col_p << 16) | row_p
    rc_m = rc_p.reshape(NW, NCH, CW)
    col_d = col_p.reshape(NW, NCH, CW)
    w_d = w_p.reshape(NW, NCH, CW)

    deg_kernel = _make_deg_kernel()
    msg128 = _make_msg_kernel(128)
    msg32 = _make_msg_kernel(32)

    deg2 = deg_kernel(col_d, w_d)
    deg2T = deg2.T

    g1 = _g1_call(x, W1, deg2T)
    S1 = msg128(g1, rc_m, w_d)
    g2 = _mid_call(S1, g1, deg2T, b1.reshape(1, 128), W2)
    S2 = msg32(g2, rc_m, w_d)
    Wcp = jnp.pad(Wc, ((0, 0), (0, 5)))
    bcp = jnp.pad(bc, (0, 5)).reshape(1, 8)
    outp = _head_call(S2, g2, deg2T, b2.reshape(1, 32), Wcp, bcp)
    return outp[:, :3]


# speedup vs baseline: 1.2869x; 1.0112x over previous
import functools

import jax
import jax.numpy as jnp
from jax import lax
from jax.experimental import pallas as pl
from jax.experimental.pallas import tpu as pltpu
from jax.experimental.pallas import tpu_sc as plsc

N = 10000
E = 320000
NC, NS, L = 2, 16, 16
NW = NC * NS
CW = 128
NCH = (E // NW + CW - 1) // CW
EWP = NCH * CW
EP = NW * EWP
NPAD = 10240
PER = NPAD // NS
BLK = 1000
GRID = N // BLK


def _make_deg_kernel():
    mesh = plsc.VectorSubcoreMesh(core_axis_name="c", subcore_axis_name="s")

    @functools.partial(
        pl.kernel, mesh=mesh,
        out_type=jax.ShapeDtypeStruct((NC, NPAD), jnp.float32),
        scratch_types=[
            pltpu.VMEM_SHARED((NPAD,), jnp.float32),
            pltpu.VMEM((NCH, CW), jnp.int32),
            pltpu.VMEM((NCH, CW), jnp.float32),
            pltpu.VMEM((PER,), jnp.float32),
            pltpu.SemaphoreType.DMA,
        ],
    )
    def deg_kernel(col_hbm, w_hbm, out_hbm, acc, col_v, w_v, init_v, dsem):
        c = lax.axis_index("c")
        s = lax.axis_index("s")
        wid = c * NS + s
        val = jnp.where(c == 0, jnp.float32(1.0), jnp.float32(0.0))
        vec = jnp.full((L,), val, jnp.float32)
        for k in range(PER // L):
            init_v[pl.ds(k * L, L)] = vec
        pltpu.sync_copy(init_v, acc.at[pl.ds(s * PER, PER)])
        plsc.subcore_barrier()
        pltpu.sync_copy(col_hbm.at[wid], col_v)
        pltpu.sync_copy(w_hbm.at[wid], w_v)

        def fire(j, carry):
            pltpu.async_copy(w_v.at[j], acc.at[col_v.at[j]], dsem, add=True)
            return carry

        lax.fori_loop(0, NCH, fire, 0)

        def drain(j, carry):
            pltpu.make_async_copy(
                w_v.at[0], acc.at[col_v.at[0]], dsem).wait()
            return carry

        lax.fori_loop(0, NCH, drain, 0)
        plsc.subcore_barrier()
        pltpu.sync_copy(acc.at[pl.ds(s * PER, PER)],
                        out_hbm.at[c, pl.ds(s * PER, PER)])

    return deg_kernel


def _make_msg_kernel(F):
    mesh = plsc.VectorSubcoreMesh(core_axis_name="c", subcore_axis_name="s")
    KV = F // L
    SW = 32 if F == 128 else 128
    NBUF = 6 if F == 128 else 8
    PF = 5 if F == 128 else 6
    SCH = EWP // SW
    MAIN = (SCH // NBUF) * NBUF
    QV = SW // L
    RPC = CW // SW
    tiled = F == 128

    @functools.partial(
        pl.kernel, mesh=mesh,
        out_type=jax.ShapeDtypeStruct((NC, NPAD, F), jnp.float32),
        scratch_types=[
            pltpu.VMEM_SHARED((NPAD, F), jnp.float32),
            pltpu.VMEM((NCH, CW), jnp.int32),
            pltpu.VMEM((NCH, CW), jnp.float32),
            pltpu.VMEM((NBUF, SW), jnp.int32),
            pltpu.VMEM((NBUF, SW), jnp.int32),
            pltpu.VMEM((NBUF, SW, F), jnp.float32),
            [pltpu.SemaphoreType.DMA] * NBUF,
            [pltpu.SemaphoreType.DMA] * NBUF,
        ],
        compiler_params=pltpu.CompilerParams(use_tc_tiling_on_sc=tiled),
    )
    def msg_kernel(g_hbm, rc_hbm, w_hbm, out_hbm,
                   acc, rc_v, w_v, ridx, cidx, rows_v, gsems, ssems):
        c = lax.axis_index("c")
        s = lax.axis_index("s")
        wid = c * NS + s
        zv = jnp.zeros((L,), jnp.float32)
        mask = jnp.full((L,), 0xFFFF, jnp.int32)

        def rcpos(j, q):
            if RPC == 1:
                return j, q * L
            return j >> 2, (j & 3) * SW + q * L

        def zrow(e, carry):
            for k in range(KV):
                rows_v[0, e, pl.ds(k * L, L)] = zv
            return carry

        lax.fori_loop(0, SW, zrow, 0)
        for t in range(PER // SW):
            pltpu.sync_copy(rows_v.at[0], acc.at[pl.ds(s * PER + t * SW, SW)])
        plsc.subcore_barrier()

        pltpu.sync_copy(rc_hbm.at[wid], rc_v)
        pltpu.sync_copy(w_hbm.at[wid], w_v)

        def unpack_rows(j, b):
            for q in range(QV):
                r, o = rcpos(j, q)
                v = rc_v[r, pl.ds(o, L)]
                ridx[b, pl.ds(q * L, L)] = v & mask

        def unpack_cols(j, b):
            for q in range(QV):
                r, o = rcpos(j, q)
                v = rc_v[r, pl.ds(o, L)]
                cidx[b, pl.ds(q * L, L)] = lax.shift_right_logical(v, 16)

        def gather(j, b):
            unpack_rows(j, b)
            pltpu.async_copy(g_hbm.at[ridx.at[b]], rows_v.at[b], gsems[b])

        def wait_gather(b):
            pltpu.make_async_copy(
                g_hbm.at[ridx.at[b]], rows_v.at[b], gsems[b]).wait()

        def scatter(j, b):
            unpack_cols(j, b)
            pltpu.async_copy(
                rows_v.at[b], acc.at[cidx.at[b]], ssems[b], add=True)

        def wait_scatter(b):
            pltpu.make_async_copy(
                rows_v.at[b], acc.at[cidx.at[b]], ssems[b]).wait()

        def scale(j, b):
            def group(gq, carry):
                r, o = rcpos(j, gq)
                wch = w_v[r, pl.ds(o, L)]
                for i in range(L):
                    wv = wch[i]
                    e = gq * L + i
                    for k in range(KV):
                        rows_v[b, e, pl.ds(k * L, L)] = (
                            rows_v[b, e, pl.ds(k * L, L)] * wv)
                return carry

            if QV > 2:
                lax.fori_loop(0, QV, group, 0)
            else:
                for gq in range(QV):
                    group(gq, 0)

        def step(j, b):
            wait_gather(b)
            scale(j, b)
            scatter(j, b)
            bn = (b + PF) % NBUF
            if isinstance(j, int):
                if j >= NBUF - PF:
                    wait_scatter(bn)
                if j + PF < SCH:
                    gather(j + PF, bn)
            else:
                @pl.when(j >= NBUF - PF)
                def _():
                    wait_scatter(bn)

                @pl.when(j + PF < SCH)
                def _():
                    gather(j + PF, bn)

        for b in range(PF):
            gather(b, b)

        def body(t, carry):
            j0 = t * NBUF
            for b in range(NBUF):
                step(j0 + b, b)
            return carry

        lax.fori_loop(0, MAIN // NBUF, body, 0)
        for j in range(MAIN, SCH):
            step(j, j % NBUF)
        for j in range(SCH - (NBUF - PF), SCH):
            wait_scatter(j % NBUF)
        plsc.subcore_barrier()
        for t in range(PER // CW):
            pltpu.sync_copy(acc.at[pl.ds(s * PER + t * CW, CW)],
                            out_hbm.at[c, pl.ds(s * PER + t * CW, CW)])

    return msg_kernel


def _dinv_block(deg_ref):
    deg = deg_ref[:, 0:1] + deg_ref[:, 1:2]
    return jnp.where(deg > 0, lax.rsqrt(deg), 0.0)


def _g1_body(x_ref, w_ref, deg_ref, out_ref):
    dinv = _dinv_block(deg_ref)
    out_ref[...] = jnp.dot(x_ref[...], w_ref[...],
                           preferred_element_type=jnp.float32) * dinv


def _g1_call(x, W1, deg2T):
    return pl.pallas_call(
        _g1_body,
        grid=(GRID,),
        in_specs=[
            pl.BlockSpec((BLK, 128), lambda i: (i, 0)),
            pl.BlockSpec((128, 128), lambda i: (0, 0)),
            pl.BlockSpec((BLK, NC), lambda i: (i, 0)),
        ],
        out_specs=pl.BlockSpec((BLK, 128), lambda i: (i, 0)),
        out_shape=jax.ShapeDtypeStruct((N, 128), jnp.float32),
    )(x, W1, deg2T)


def _leaky(h):
    return jnp.where(h > 0, h, 0.01 * h)


def _mid_body(s_ref, g_ref, deg_ref, b_ref, w2_ref, out_ref):
    dinv = _dinv_block(deg_ref)
    a1 = _leaky(dinv * (s_ref[0] + s_ref[1] + g_ref[...]) + b_ref[...])
    out_ref[...] = jnp.dot(a1, w2_ref[...],
                           preferred_element_type=jnp.float32) * dinv


def _mid_call(S1, g1, deg2T, b1r, W2):
    return pl.pallas_call(
        _mid_body,
        grid=(GRID,),
        in_specs=[
            pl.BlockSpec((NC, BLK, 128), lambda i: (0, i, 0)),
            pl.BlockSpec((BLK, 128), lambda i: (i, 0)),
            pl.BlockSpec((BLK, NC), lambda i: (i, 0)),
            pl.BlockSpec((1, 128), lambda i: (0, 0)),
            pl.BlockSpec((128, 32), lambda i: (0, 0)),
        ],
        out_specs=pl.BlockSpec((BLK, 32), lambda i: (i, 0)),
        out_shape=jax.ShapeDtypeStruct((N, 32), jnp.float32),
    )(S1, g1, deg2T, b1r, W2)


def _head_body(s_ref, g_ref, deg_ref, b_ref, wc_ref, bc_ref, out_ref):
    dinv = _dinv_block(deg_ref)
    a2 = _leaky(dinv * (s_ref[0] + s_ref[1] + g_ref[...]) + b_ref[...])
    out_ref[...] = jnp.dot(a2, wc_ref[...],
                           preferred_element_type=jnp.float32) + bc_ref[...]


def _head_call(S2, g2, deg2T, b2r, Wcp, bcp):
    return pl.pallas_call(
        _head_body,
        grid=(GRID,),
        in_specs=[
            pl.BlockSpec((NC, BLK, 32), lambda i: (0, i, 0)),
            pl.BlockSpec((BLK, 32), lambda i: (i, 0)),
            pl.BlockSpec((BLK, NC), lambda i: (i, 0)),
            pl.BlockSpec((1, 32), lambda i: (0, 0)),
            pl.BlockSpec((32, 8), lambda i: (0, 0)),
            pl.BlockSpec((1, 8), lambda i: (0, 0)),
        ],
        out_specs=pl.BlockSpec((BLK, 8), lambda i: (i, 0)),
        out_shape=jax.ShapeDtypeStruct((N, 8), jnp.float32),
    )(S2, g2, deg2T, b2r, Wcp, bcp)


def kernel(x, edge_index, edge_attr, W1, b1, W2, b2, Wc, bc):
    row = edge_index[0]
    col = edge_index[1]
    pade = EP - E
    pad_idx = (jnp.arange(pade, dtype=jnp.int32) * 97) % N
    row_p = jnp.concatenate([row, pad_idx])
    col_p = jnp.concatenate([col, pad_idx])
    w_p = jnp.concatenate([edge_attr, jnp.zeros((pade,), jnp.float32)])
    rc_p = (col_p << 16) | row_p
    rc_m = rc_p.reshape(NW, NCH, CW)
    col_d = col_p.reshape(NW, NCH, CW)
    w_d = w_p.reshape(NW, NCH, CW)

    deg_kernel = _make_deg_kernel()
    msg128 = _make_msg_kernel(128)
    msg32 = _make_msg_kernel(32)

    deg2 = deg_kernel(col_d, w_d)
    deg2T = deg2.T

    g1 = _g1_call(x, W1, deg2T)
    S1 = msg128(g1, rc_m, w_d)
    g2 = _mid_call(S1, g1, deg2T, b1.reshape(1, 128), W2)
    S2 = msg32(g2, rc_m, w_d)
    Wcp = jnp.pad(Wc, ((0, 0), (0, 5)))
    bcp = jnp.pad(bc, (0, 5)).reshape(1, 8)
    outp = _head_call(S2, g2, deg2T, b2.reshape(1, 32), Wcp, bcp)
    return outp[:, :3]
